# baseline (speedup 1.0000x reference)
import functools

import jax
import jax.numpy as jnp
from jax import lax
from jax.experimental import pallas as pl
from jax.experimental.pallas import tpu as pltpu
from jax.experimental.pallas import tpu_sc as plsc

_B, _S, _D, _K = 16, 576, 64, 1024
_CC = 0.99


def _row_sum(sq):
    acc = sq[:, 0:8]
    for r in range(1, 8):
        acc = acc + sq[:, 8 * r:8 * r + 8]
    for w in (4, 2, 1):
        acc = acc[:, :w] + acc[:, w:2 * w]
    return acc


def _stage_a_body(inp_ref, emb_ref, x_ref, rank_ref, enc_ref):
    xin = inp_ref[0]
    scale = jnp.sqrt(_row_sum(xin ** 2))
    x = xin / scale
    x_ref[0] = x
    e = emb_ref[...]
    d = (_row_sum(x ** 2)
         + _row_sum(e ** 2)[:, 0]
         - 2.0 * lax.dot_general(x, e, (((1,), (1,)), ((), ()))))
    md = jnp.min(d, axis=1)
    iota_k = lax.broadcasted_iota(jnp.int32, (_S, _K), 1)
    mdc = md[:, None]
    enc = jnp.min(jnp.where(d == mdc, iota_k, _K), axis=1)
    mdr = md[None, :]
    jr = lax.broadcasted_iota(jnp.int32, (1, _S), 1)
    rank = jnp.zeros((1, _S), jnp.int32)
    _CH = 64
    for c in range(_S // _CH):
        mi = lax.slice(mdc, (c * _CH, 0), ((c + 1) * _CH, 1))
        ii = lax.broadcasted_iota(jnp.int32, (_CH, 1), 0) + c * _CH
        cmp = (mi < mdr) | ((mi == mdr) & (ii < jr))
        rank = rank + jnp.sum(cmp.astype(jnp.int32), axis=0, keepdims=True)
    rank_ref[0, 0] = rank[0]
    enc_ref[0, 0] = enc


def _stage_a(inputs, embedding):
    return pl.pallas_call(
        _stage_a_body,
        grid=(_B,),
        in_specs=[
            pl.BlockSpec((1, _S, _D), lambda b: (b, 0, 0)),
            pl.BlockSpec((_K, _D), lambda b: (0, 0)),
        ],
        out_specs=[
            pl.BlockSpec((1, _S, _D), lambda b: (b, 0, 0)),
            pl.BlockSpec((1, 1, _S), lambda b: (b, 0, 0)),
            pl.BlockSpec((1, 1, _S), lambda b: (b, 0, 0)),
        ],
        out_shape=[
            jax.ShapeDtypeStruct((_B, _S, _D), jnp.float32),
            jax.ShapeDtypeStruct((_B, 1, _S), jnp.int32),
            jax.ShapeDtypeStruct((_B, 1, _S), jnp.int32),
        ],
    )(inputs, embedding)


_HALF = _S // 2
_GCH = 96


def _stage_b(embedding, enc0, rank):
    mesh = plsc.VectorSubcoreMesh(core_axis_name="c", subcore_axis_name="s")

    @functools.partial(
        pl.kernel,
        mesh=mesh,
        compiler_params=pltpu.CompilerParams(
            needs_layout_passes=False, use_tc_tiling_on_sc=False),
        out_type=jax.ShapeDtypeStruct((_B * _S, _D), jnp.float32),
        scratch_types=[
            pltpu.VMEM((_S,), jnp.int32),
            pltpu.VMEM((_S,), jnp.int32),
            pltpu.VMEM((_S,), jnp.int32),
            pltpu.VMEM((_GCH, _D), jnp.float32),
            pltpu.SemaphoreType.DMA,
        ],
    )
    def sc_kernel(emb_hbm, enc_hbm, rank_hbm, out_hbm,
                  enc_v, rank_v, fidx_v, rows_v, sem):
        wid = lax.axis_index("s") * 2 + lax.axis_index("c")
        b = wid // 2
        h = wid % 2
        pltpu.sync_copy(enc_hbm, enc_v)
        pltpu.sync_copy(rank_hbm.at[b], rank_v)

        def scatter_body(i, carry):
            sl = pl.ds(i * 16, 16)
            plsc.store_scatter(fidx_v, [rank_v[sl]], enc_v[sl])
            return carry

        lax.fori_loop(0, _S // 16, scatter_body, 0)

        def gather_body(g, carry):
            srow = h * _HALF + g * _GCH
            cp = pltpu.async_copy(
                emb_hbm.at[fidx_v.at[pl.ds(srow, _GCH)]], rows_v, sem)
            cp.wait()
            pltpu.sync_copy(rows_v, out_hbm.at[pl.ds(b * _S + srow, _GCH)])
            return carry

        lax.fori_loop(0, _HALF // _GCH, gather_body, 0)

    return sc_kernel(embedding, enc0, rank)


def _stage_c_body(q_ref, x_ref, qst_ref, loss_ref):
    qq = q_ref[...]
    xx = x_ref[...]
    e_latent = jnp.mean((qq - xx) ** 2)
    q_latent = jnp.mean((qq - xx) ** 2)
    loss_ref[...] = (q_latent + _CC * e_latent).reshape(1, 1)
    qst_ref[...] = xx + (qq - xx)


def _stage_c(q, x):
    return pl.pallas_call(
        _stage_c_body,
        out_shape=[
            jax.ShapeDtypeStruct((_B, _S, _D), jnp.float32),
            jax.ShapeDtypeStruct((1, 1), jnp.float32),
        ],
    )(q, x)


def kernel(inputs, embedding):
    x, rank3, enc3 = _stage_a(inputs, embedding)
    enc0 = enc3[0, 0]
    rank = rank3[:, 0, :]
    qflat = _stage_b(embedding, enc0, rank)
    qst, loss = _stage_c(qflat.reshape(_B, _S, _D), x)
    return qst, loss[0, 0]
